# baseline (speedup 1.0000x reference)
import functools

import jax
import jax.numpy as jnp
from jax import lax
from jax.experimental import pallas as pl
from jax.experimental.pallas import tpu as pltpu
from jax.experimental.pallas import tpu_sc as plsc

N_NODES = 100000
FEAT = 128
NW = 32
CH = 16
ROWS = 7 * CH
CPW = 195
UNROLL = 4

_mesh = plsc.VectorSubcoreMesh(core_axis_name="c", subcore_axis_name="s")


@functools.partial(
    pl.kernel,
    mesh=_mesh,
    compiler_params=pltpu.CompilerParams(needs_layout_passes=False),
    out_type=[
        jax.ShapeDtypeStruct((N_NODES, FEAT), jnp.float32),
        jax.ShapeDtypeStruct((N_NODES, FEAT), jnp.int32),
    ],
    scratch_types=[
        pltpu.VMEM((ROWS,), jnp.int32),
        pltpu.VMEM((ROWS, FEAT), jnp.float32),
        pltpu.VMEM((CH, FEAT), jnp.float32),
        pltpu.VMEM((CH, FEAT), jnp.int32),
        pltpu.SemaphoreType.DMA,
    ],
)
def _sc_pool(x_hbm, no_hbm, vals_hbm, idxs_hbm, idx_v, rows_v, vout_v, iout_v, sem):
    wid = lax.axis_index("s") * 2 + lax.axis_index("c")
    node0 = CH * CPW * wid + CH * jnp.minimum(wid, 10)
    n_chunks = jnp.where(wid < 10, CPW + 1, CPW)

    iota = lax.iota(jnp.int32, 16)
    iota7 = iota * 7
    kf = [jnp.full((16,), float(k), jnp.float32) for k in range(7)]

    def chunk_body(g, _):
        node_base = node0 + g * CH
        pltpu.sync_copy(no_hbm.at[pl.ds(node_base * 7, ROWS)], idx_v)
        pltpu.async_copy(x_hbm.at[idx_v], rows_v, sem).wait()

        def f_block(i, _):
            for j in range(UNROLL):
                f = i * UNROLL + j
                p0 = f * 7
                bval = None
                bidx = None
                for k in range(7):
                    p = p0 + k
                    r = p >> 7
                    c = p & 127
                    col = jnp.full((16,), c, jnp.int32)
                    gv = plsc.load_gather(rows_v, [iota7 + r, col])
                    if k == 0:
                        bval = gv
                        bidx = kf[0]
                    else:
                        m = gv > bval
                        bval = jnp.maximum(bval, gv)
                        bidx = jnp.where(m, kf[k], bidx)
                colf = jnp.full((16,), f, jnp.int32)
                plsc.store_scatter(vout_v, [iota, colf], bval)
                plsc.store_scatter(iout_v, [iota, colf], bidx.astype(jnp.int32))
            return 0

        lax.fori_loop(0, FEAT // UNROLL, f_block, 0)
        pltpu.sync_copy(vout_v, vals_hbm.at[pl.ds(node_base, CH)])
        pltpu.sync_copy(iout_v, idxs_hbm.at[pl.ds(node_base, CH)])
        return 0

    lax.fori_loop(0, n_chunks, chunk_body, 0)


def kernel(x, neigh_orders):
    no32 = neigh_orders.astype(jnp.int32)
    vals, idxs = _sc_pool(x, no32)
    return (vals, idxs)
